# baseline (speedup 1.0000x reference)
import functools

import numpy as np
import jax
import jax.numpy as jnp
from jax.experimental import pallas as pl
from jax.experimental.pallas import tpu as pltpu


@functools.lru_cache(maxsize=None)
def _bicubic_matrix(in_size: int, out_size: int) -> np.ndarray:
    a = -0.75
    scale = in_size / out_size
    i = np.arange(out_size, dtype=np.float64)
    real = scale * (i + 0.5) - 0.5
    idx0 = np.floor(real)
    t = real - idx0

    def cubic1(x):
        return ((a + 2.0) * x - (a + 3.0)) * x * x + 1.0

    def cubic2(x):
        return ((a * x - 5.0 * a) * x + 8.0 * a) * x - 4.0 * a

    w = np.stack([cubic2(t + 1.0), cubic1(t), cubic1(1.0 - t), cubic2(2.0 - t)],
                 axis=1)
    idx = idx0.astype(np.int64)[:, None] + np.arange(-1, 3)[None, :]
    idx = np.clip(idx, 0, in_size - 1)
    W = np.zeros((out_size, in_size), dtype=np.float64)
    rows = np.arange(out_size)
    for k in range(4):
        np.add.at(W, (rows, idx[:, k]), w[:, k])
    return W


def _nhwc_kernel(x_ref, ww_ref, wh_ref, o_ref, zx_ref, *, h_in, w_in, c,
                 h_out, w_out, i_blk):
    t = pl.program_id(1)

    @pl.when(t == 0)
    def _height_pass():
        x2d = x_ref[0].astype(jnp.bfloat16).reshape(h_in, w_in * c)
        zh = jax.lax.dot_general(wh_ref[...], x2d, (((1,), (0,)), ((), ())),
                                 preferred_element_type=jnp.float32)
        zhb = zh.astype(jnp.bfloat16)
        zx_ref[...] = zhb.reshape(h_out, w_in, c).transpose(1, 0, 2) \
                         .reshape(w_in, h_out * c)

    ww = ww_ref[...]
    for it in range(i_blk // 8):
        seg = zx_ref[:, pl.ds((t * i_blk + it * 8) * c, 8 * c)]
        og = jax.lax.dot_general(ww, seg, (((1,), (0,)), ((), ())),
                                 preferred_element_type=jnp.float32)
        for d in range(8):
            o_ref[0, it * 8 + d] = og[:, d * c:(d + 1) * c]


def _resize_nhwc(x_nhwc, h_out, w_out):
    n, h_in, w_in, c = x_nhwc.shape
    i_blk = h_out // 2

    wh = jnp.asarray(_bicubic_matrix(h_in, h_out), dtype=jnp.bfloat16)
    ww = jnp.asarray(_bicubic_matrix(w_in, w_out), dtype=jnp.bfloat16)

    flops = 2 * n * c * (h_out * h_in * w_in + h_out * w_in * w_out)
    bytes_accessed = 4 * n * c * (h_in * w_in + h_out * w_out)

    body = functools.partial(_nhwc_kernel, h_in=h_in, w_in=w_in, c=c,
                             h_out=h_out, w_out=w_out, i_blk=i_blk)
    return pl.pallas_call(
        body,
        out_shape=jax.ShapeDtypeStruct((n, h_out, w_out, c), x_nhwc.dtype),
        grid=(n, h_out // i_blk),
        in_specs=[
            pl.BlockSpec((1, h_in, w_in, c), lambda i, t: (i, 0, 0, 0)),
            pl.BlockSpec(ww.shape, lambda i, t: (0, 0)),
            pl.BlockSpec(wh.shape, lambda i, t: (0, 0)),
        ],
        out_specs=pl.BlockSpec((1, i_blk, w_out, c),
                               lambda i, t: (i, t, 0, 0)),
        scratch_shapes=[pltpu.VMEM((w_in, h_out * c), jnp.bfloat16)],
        compiler_params=pltpu.CompilerParams(
            dimension_semantics=("parallel", "arbitrary"),
            vmem_limit_bytes=96 * 1024 * 1024,
        ),
        cost_estimate=pl.CostEstimate(
            flops=flops, transcendentals=0, bytes_accessed=bytes_accessed),
    )(x_nhwc, ww, wh)


_CONTRACT0 = (((0,), (0,)), ((), ()))


@functools.lru_cache(maxsize=None)
def _planar_weights(h_in, w_in, h_out, w_out, chunk):
    wh = _bicubic_matrix(h_in, h_out)
    ww = _bicubic_matrix(w_in, w_out)
    whd_t = np.kron(np.eye(chunk), wh.T)
    return whd_t.astype(np.float32), ww.T.astype(np.float32)


def _planar_kernel(x_ref, whd_ref, wwt_ref, o_ref, *, chunk, h_in, h_out):
    b, _, w_in = x_ref.shape
    w_out = o_ref.shape[2]
    whd = whd_ref[...]
    wwt = wwt_ref[...]
    for cc in range(b // chunk):
        xc = x_ref[cc * chunk:(cc + 1) * chunk].astype(jnp.bfloat16)
        zt = jax.lax.dot_general(xc.reshape(chunk * h_in, w_in), whd,
                                 _CONTRACT0, preferred_element_type=jnp.float32)
        oc = jax.lax.dot_general(zt.astype(jnp.bfloat16), wwt, _CONTRACT0,
                                 preferred_element_type=jnp.float32)
        o_ref[cc * chunk:(cc + 1) * chunk] = oc.reshape(chunk, h_out, w_out)


def _resize_planar(x3d, h_out, w_out):
    nc, h_in, w_in = x3d.shape
    chunk = 8
    while chunk > 1 and nc % chunk:
        chunk //= 2
    block = next(chunk * m for m in (4, 2, 1) if nc % (chunk * m) == 0)

    whd_np, wwt_np = _planar_weights(h_in, w_in, h_out, w_out, chunk)
    whd = jnp.asarray(whd_np, dtype=jnp.bfloat16)
    wwt = jnp.asarray(wwt_np, dtype=jnp.bfloat16)

    flops = 2 * nc * (h_out * h_in * w_in + h_out * w_in * w_out)
    bytes_accessed = 4 * nc * (h_in * w_in + h_out * w_out)

    body = functools.partial(_planar_kernel, chunk=chunk, h_in=h_in,
                             h_out=h_out)
    return pl.pallas_call(
        body,
        out_shape=jax.ShapeDtypeStruct((nc, h_out, w_out), x3d.dtype),
        grid=(nc // block,),
        in_specs=[
            pl.BlockSpec((block, h_in, w_in), lambda i: (i, 0, 0)),
            pl.BlockSpec(whd.shape, lambda i: (0, 0)),
            pl.BlockSpec(wwt.shape, lambda i: (0, 0)),
        ],
        out_specs=pl.BlockSpec((block, h_out, w_out), lambda i: (i, 0, 0)),
        compiler_params=pltpu.CompilerParams(
            dimension_semantics=("parallel",),
            vmem_limit_bytes=96 * 1024 * 1024,
        ),
        cost_estimate=pl.CostEstimate(
            flops=flops, transcendentals=0, bytes_accessed=bytes_accessed),
    )(x3d, whd, wwt)


def kernel(lq):
    n, c, h_in, w_in = lq.shape
    h_out = w_out = 224
    if c % 128 == 0 and h_in % 8 == 0:
        x_nhwc = jnp.transpose(lq, (0, 2, 3, 1))
        out = _resize_nhwc(x_nhwc, h_out, w_out)
        return jnp.transpose(out, (0, 3, 1, 2))
    out3d = _resize_planar(lq.reshape(n * c, h_in, w_in), h_out, w_out)
    return out3d.reshape(n, c, h_out, w_out)


# speedup vs baseline: 1.2445x; 1.2445x over previous
import functools

import numpy as np
import jax
import jax.numpy as jnp
from jax.experimental import pallas as pl
from jax.experimental.pallas import tpu as pltpu


@functools.lru_cache(maxsize=None)
def _bicubic_matrix(in_size: int, out_size: int) -> np.ndarray:
    a = -0.75
    scale = in_size / out_size
    i = np.arange(out_size, dtype=np.float64)
    real = scale * (i + 0.5) - 0.5
    idx0 = np.floor(real)
    t = real - idx0

    def cubic1(x):
        return ((a + 2.0) * x - (a + 3.0)) * x * x + 1.0

    def cubic2(x):
        return ((a * x - 5.0 * a) * x + 8.0 * a) * x - 4.0 * a

    w = np.stack([cubic2(t + 1.0), cubic1(t), cubic1(1.0 - t), cubic2(2.0 - t)],
                 axis=1)
    idx = idx0.astype(np.int64)[:, None] + np.arange(-1, 3)[None, :]
    idx = np.clip(idx, 0, in_size - 1)
    W = np.zeros((out_size, in_size), dtype=np.float64)
    rows = np.arange(out_size)
    for k in range(4):
        np.add.at(W, (rows, idx[:, k]), w[:, k])
    return W


def _nhwc_kernel(x_ref, ww_ref, wh_ref, o_ref, *, h_in, w_in, c, h_out,
                 w_out):
    x2d = x_ref[0].astype(jnp.bfloat16).reshape(h_in, w_in * c)
    zh = jax.lax.dot_general(wh_ref[...], x2d, (((1,), (0,)), ((), ())),
                             preferred_element_type=jnp.float32)
    zhb = zh.astype(jnp.bfloat16)
    zx = zhb.reshape(h_out, w_in, c).transpose(1, 0, 2).reshape(w_in,
                                                                h_out * c)
    ww = ww_ref[...]
    for it in range(h_out // 8):
        seg = zx[:, it * 8 * c:(it + 1) * 8 * c]
        og = jax.lax.dot_general(ww, seg, (((1,), (0,)), ((), ())),
                                 preferred_element_type=jnp.float32)
        for d in range(8):
            o_ref[0, it * 8 + d] = og[:, d * c:(d + 1) * c]


def _resize_nhwc(x_nhwc, h_out, w_out):
    n, h_in, w_in, c = x_nhwc.shape

    wh = jnp.asarray(_bicubic_matrix(h_in, h_out), dtype=jnp.bfloat16)
    ww = jnp.asarray(_bicubic_matrix(w_in, w_out), dtype=jnp.bfloat16)

    flops = 2 * n * c * (h_out * h_in * w_in + h_out * w_in * w_out)
    bytes_accessed = 4 * n * c * (h_in * w_in + h_out * w_out)

    body = functools.partial(_nhwc_kernel, h_in=h_in, w_in=w_in, c=c,
                             h_out=h_out, w_out=w_out)
    return pl.pallas_call(
        body,
        out_shape=jax.ShapeDtypeStruct((n, h_out, w_out, c), x_nhwc.dtype),
        grid=(n,),
        in_specs=[
            pl.BlockSpec((1, h_in, w_in, c), lambda i: (i, 0, 0, 0)),
            pl.BlockSpec(ww.shape, lambda i: (0, 0)),
            pl.BlockSpec(wh.shape, lambda i: (0, 0)),
        ],
        out_specs=pl.BlockSpec((1, h_out, w_out, c), lambda i: (i, 0, 0, 0)),
        compiler_params=pltpu.CompilerParams(
            dimension_semantics=("parallel",),
            vmem_limit_bytes=96 * 1024 * 1024,
        ),
        cost_estimate=pl.CostEstimate(
            flops=flops, transcendentals=0, bytes_accessed=bytes_accessed),
    )(x_nhwc, ww, wh)


_CONTRACT0 = (((0,), (0,)), ((), ()))


@functools.lru_cache(maxsize=None)
def _planar_weights(h_in, w_in, h_out, w_out, chunk):
    wh = _bicubic_matrix(h_in, h_out)
    ww = _bicubic_matrix(w_in, w_out)
    whd_t = np.kron(np.eye(chunk), wh.T)
    return whd_t.astype(np.float32), ww.T.astype(np.float32)


def _planar_kernel(x_ref, whd_ref, wwt_ref, o_ref, *, chunk, h_in, h_out):
    b, _, w_in = x_ref.shape
    w_out = o_ref.shape[2]
    whd = whd_ref[...]
    wwt = wwt_ref[...]
    for cc in range(b // chunk):
        xc = x_ref[cc * chunk:(cc + 1) * chunk].astype(jnp.bfloat16)
        zt = jax.lax.dot_general(xc.reshape(chunk * h_in, w_in), whd,
                                 _CONTRACT0, preferred_element_type=jnp.float32)
        oc = jax.lax.dot_general(zt.astype(jnp.bfloat16), wwt, _CONTRACT0,
                                 preferred_element_type=jnp.float32)
        o_ref[cc * chunk:(cc + 1) * chunk] = oc.reshape(chunk, h_out, w_out)


def _resize_planar(x3d, h_out, w_out):
    nc, h_in, w_in = x3d.shape
    chunk = 8
    while chunk > 1 and nc % chunk:
        chunk //= 2
    block = next(chunk * m for m in (4, 2, 1) if nc % (chunk * m) == 0)

    whd_np, wwt_np = _planar_weights(h_in, w_in, h_out, w_out, chunk)
    whd = jnp.asarray(whd_np, dtype=jnp.bfloat16)
    wwt = jnp.asarray(wwt_np, dtype=jnp.bfloat16)

    flops = 2 * nc * (h_out * h_in * w_in + h_out * w_in * w_out)
    bytes_accessed = 4 * nc * (h_in * w_in + h_out * w_out)

    body = functools.partial(_planar_kernel, chunk=chunk, h_in=h_in,
                             h_out=h_out)
    return pl.pallas_call(
        body,
        out_shape=jax.ShapeDtypeStruct((nc, h_out, w_out), x3d.dtype),
        grid=(nc // block,),
        in_specs=[
            pl.BlockSpec((block, h_in, w_in), lambda i: (i, 0, 0)),
            pl.BlockSpec(whd.shape, lambda i: (0, 0)),
            pl.BlockSpec(wwt.shape, lambda i: (0, 0)),
        ],
        out_specs=pl.BlockSpec((block, h_out, w_out), lambda i: (i, 0, 0)),
        compiler_params=pltpu.CompilerParams(
            dimension_semantics=("parallel",),
            vmem_limit_bytes=96 * 1024 * 1024,
        ),
        cost_estimate=pl.CostEstimate(
            flops=flops, transcendentals=0, bytes_accessed=bytes_accessed),
    )(x3d, whd, wwt)


def kernel(lq):
    n, c, h_in, w_in = lq.shape
    h_out = w_out = 224
    if c % 128 == 0 and h_in % 8 == 0:
        x_nhwc = jnp.transpose(lq, (0, 2, 3, 1))
        out = _resize_nhwc(x_nhwc, h_out, w_out)
        return jnp.transpose(out, (0, 3, 1, 2))
    out3d = _resize_planar(lq.reshape(n * c, h_in, w_in), h_out, w_out)
    return out3d.reshape(n, c, h_out, w_out)
